# baseline (speedup 1.0000x reference)
import functools

import jax
import jax.numpy as jnp
from jax import lax
from jax.experimental import pallas as pl
from jax.experimental.pallas import tpu as pltpu
from jax.experimental.pallas import tpu_sc as plsc

N_TOK = 32768
D = 4096
E = 64
K = 8
BLK = 1024

NC = 2
NS = 16
NW = NC * NS
CH = N_TOK // NW
GRP = 16

NEG_INF = float("-inf")


def _scores_kernel(w_ref, x_ref, eps_ref, s3_ref):
    w = w_ref[...]
    x = x_ref[...]
    s2 = jax.lax.dot_general(
        w, x, (((1,), (1,)), ((), ())), preferred_element_type=jnp.float32
    )
    gate = s2[:E, :]
    noise_std = jax.nn.softplus(s2[E:, :])
    s3_ref[0] = gate + noise_std * eps_ref[...]


def _tc_scores(x, wcat, eps_t):
    grid = (N_TOK // BLK,)
    return pl.pallas_call(
        _scores_kernel,
        grid=grid,
        in_specs=[
            pl.BlockSpec((2 * E, D), lambda i: (0, 0)),
            pl.BlockSpec((BLK, D), lambda i: (i, 0)),
            pl.BlockSpec((E, BLK), lambda i: (0, i)),
        ],
        out_specs=pl.BlockSpec((1, E, BLK), lambda i: (i, 0, 0)),
        out_shape=jax.ShapeDtypeStruct((NW, E, CH), jnp.float32),
    )(wcat, x, eps_t)


@functools.partial(
    pl.kernel,
    out_type=[
        jax.ShapeDtypeStruct((NW, K, CH), jnp.float32),
        jax.ShapeDtypeStruct((NW, K, CH), jnp.int32),
    ],
    mesh=plsc.VectorSubcoreMesh(core_axis_name="c", subcore_axis_name="s"),
    scratch_types=[
        pltpu.VMEM((E, CH), jnp.float32),
        pltpu.VMEM((K, CH), jnp.float32),
        pltpu.VMEM((K, CH), jnp.int32),
    ],
)
def _sc_topk(s3_hbm, pw_hbm, pi_hbm, s_v, w_v, i_v):
    wid = lax.axis_index("s") * NC + lax.axis_index("c")
    pltpu.sync_copy(s3_hbm.at[wid], s_v)
    neg_inf = jnp.full((GRP,), NEG_INF, jnp.float32)
    zero_i = jnp.zeros((GRP,), jnp.int32)
    one_i = jnp.full((GRP,), 1, jnp.int32)

    def group(g, carry):
        base = g * GRP
        sl = pl.ds(base, GRP)
        sel_lo = zero_i
        sel_hi = zero_i
        vals = []
        idxs = []
        for r in range(K):
            m = neg_inf
            mi = zero_i
            for e in range(E):
                v = s_v[e, sl]
                if r > 0:
                    if e < 32:
                        bit = lax.shift_right_logical(sel_lo, e) & 1
                    else:
                        bit = lax.shift_right_logical(sel_hi, e - 32) & 1
                    v = jnp.where(bit != 0, neg_inf, v)
                gt = v > m
                m = jnp.where(gt, v, m)
                mi = jnp.where(gt, e, mi)
            vals.append(m)
            idxs.append(mi)
            if r < K - 1:
                b = lax.shift_left(one_i, mi & 31)
                sel_lo = sel_lo | jnp.where(mi < 32, b, zero_i)
                sel_hi = sel_hi | jnp.where(mi >= 32, b, zero_i)
        m0 = vals[0]
        es = [jnp.exp(v - m0) for v in vals]
        tot = es[0]
        for t in es[1:]:
            tot = tot + t
        for r in range(K):
            w_v[r, sl] = es[r] / tot
            i_v[r, sl] = idxs[r]
        return carry

    lax.fori_loop(0, CH // GRP, group, 0)
    pltpu.sync_copy(w_v, pw_hbm.at[wid])
    pltpu.sync_copy(i_v, pi_hbm.at[wid])


@jax.jit
def _run(x, wcat, eps_t):
    s3 = _tc_scores(x, wcat, eps_t)
    pw3, pi3 = _sc_topk(s3)
    pw = pw3.transpose(0, 2, 1).reshape(N_TOK, K)
    pi = pi3.transpose(0, 2, 1).reshape(N_TOK, K)
    return pw, pi


_CONST_CACHE = []


def _consts():
    if not _CONST_CACHE:
        eps_t = jax.jit(
            lambda: jnp.transpose(
                jax.random.normal(jax.random.key(42), (N_TOK, E), dtype=jnp.float32)
            )
        )()
        _CONST_CACHE.append(jax.block_until_ready(eps_t))
    return _CONST_CACHE[0]


def kernel(x, W_gate, W_noise):
    wcat = jnp.concatenate([W_gate, W_noise], axis=0)
    return _run(x, wcat, _consts())


# speedup vs baseline: 1.1677x; 1.1677x over previous
import functools

import jax
import jax.numpy as jnp
from jax import lax
from jax.experimental import pallas as pl
from jax.experimental.pallas import tpu as pltpu
from jax.experimental.pallas import tpu_sc as plsc

N_TOK = 32768
D = 4096
E = 64
K = 8
BLK = 1024

NC = 2
NS = 16
NW = NC * NS
GRP = 16

SC_TOK = 8192
TC_TOK = N_TOK - SC_TOK
CH = SC_TOK // NW
SC_BLOCKS = SC_TOK // BLK
TPB = NW // SC_BLOCKS

NEG_INF = float("-inf")


def _scores_kernel(w_ref, x_ref, eps_ref, s3_ref):
    w = w_ref[...]
    x = x_ref[...]
    s2 = jax.lax.dot_general(
        w, x, (((1,), (1,)), ((), ())), preferred_element_type=jnp.float32
    )
    gate = s2[:E, :]
    noise_std = jax.nn.softplus(s2[E:, :])
    s = gate + noise_std * eps_ref[...]
    for k in range(TPB):
        s3_ref[k] = s[:, k * CH:(k + 1) * CH]


def _tc_scores(x, wcat, eps_t):
    return pl.pallas_call(
        _scores_kernel,
        grid=(SC_BLOCKS,),
        in_specs=[
            pl.BlockSpec((2 * E, D), lambda i: (0, 0)),
            pl.BlockSpec((BLK, D), lambda i: (i, 0)),
            pl.BlockSpec((E, BLK), lambda i: (0, i)),
        ],
        out_specs=pl.BlockSpec((TPB, E, CH), lambda i: (i, 0, 0)),
        out_shape=jax.ShapeDtypeStruct((NW, E, CH), jnp.float32),
    )(wcat, x, eps_t)


@functools.partial(
    pl.kernel,
    out_type=[
        jax.ShapeDtypeStruct((NW, K, CH), jnp.float32),
        jax.ShapeDtypeStruct((NW, K, CH), jnp.int32),
    ],
    mesh=plsc.VectorSubcoreMesh(core_axis_name="c", subcore_axis_name="s"),
    scratch_types=[
        pltpu.VMEM((E, CH), jnp.float32),
        pltpu.VMEM((K, CH), jnp.float32),
        pltpu.VMEM((K, CH), jnp.int32),
    ],
)
def _sc_topk(s3_hbm, pw_hbm, pi_hbm, s_v, w_v, i_v):
    wid = lax.axis_index("s") * NC + lax.axis_index("c")
    pltpu.sync_copy(s3_hbm.at[wid], s_v)
    neg_inf = jnp.full((GRP,), NEG_INF, jnp.float32)
    zero_i = jnp.zeros((GRP,), jnp.int32)
    one_i = jnp.full((GRP,), 1, jnp.int32)

    def group(g, carry):
        sl = pl.ds(g * GRP, GRP)
        sel_lo = zero_i
        sel_hi = zero_i
        vals = []
        idxs = []
        for r in range(K):
            m = neg_inf
            mi = zero_i
            for e in range(E):
                v = s_v[e, sl]
                if r > 0:
                    if e < 32:
                        bit = lax.shift_right_logical(sel_lo, e) & 1
                    else:
                        bit = lax.shift_right_logical(sel_hi, e - 32) & 1
                    v = jnp.where(bit != 0, neg_inf, v)
                gt = v > m
                m = jnp.where(gt, v, m)
                mi = jnp.where(gt, e, mi)
            vals.append(m)
            idxs.append(mi)
            if r < K - 1:
                b = lax.shift_left(one_i, mi & 31)
                sel_lo = sel_lo | jnp.where(mi < 32, b, zero_i)
                sel_hi = sel_hi | jnp.where(mi >= 32, b, zero_i)
        m0 = vals[0]
        es = [jnp.exp(v - m0) for v in vals]
        tot = es[0]
        for t in es[1:]:
            tot = tot + t
        for r in range(K):
            w_v[r, sl] = es[r] / tot
            i_v[r, sl] = idxs[r]
        return carry

    lax.fori_loop(0, CH // GRP, group, 0)
    pltpu.sync_copy(w_v, pw_hbm.at[wid])
    pltpu.sync_copy(i_v, pi_hbm.at[wid])


def _fused_kernel(w_ref, x_ref, eps_ref, pw_ref, pi_ref):
    w = w_ref[...]
    x = x_ref[...]
    s2 = jax.lax.dot_general(
        w, x, (((1,), (1,)), ((), ())), preferred_element_type=jnp.float32
    )
    gate = s2[:E, :]
    noise_std = jax.nn.softplus(s2[E:, :])
    s = gate + noise_std * eps_ref[...]

    iota0 = jax.lax.broadcasted_iota(jnp.int32, (E, BLK), 0)
    vals = []
    idxs = []
    cur = s
    for _ in range(K):
        m = jnp.max(cur, axis=0, keepdims=True)
        idx = jnp.min(jnp.where(cur == m, iota0, E), axis=0, keepdims=True)
        vals.append(m)
        idxs.append(idx)
        cur = jnp.where(iota0 == idx, NEG_INF, cur)
    w8 = jnp.concatenate(vals, axis=0)
    i8 = jnp.concatenate(idxs, axis=0)
    e8 = jnp.exp(w8 - w8[0:1, :])
    p8 = e8 / jnp.sum(e8, axis=0, keepdims=True)
    pw_ref[...] = p8
    pi_ref[...] = i8


def _tc_fused(x, wcat, eps_t):
    off = SC_TOK // BLK
    return pl.pallas_call(
        _fused_kernel,
        grid=(TC_TOK // BLK,),
        in_specs=[
            pl.BlockSpec((2 * E, D), lambda i: (0, 0)),
            pl.BlockSpec((BLK, D), lambda i: (i + off, 0)),
            pl.BlockSpec((E, BLK), lambda i: (0, i + off)),
        ],
        out_specs=[
            pl.BlockSpec((K, BLK), lambda i: (0, i)),
            pl.BlockSpec((K, BLK), lambda i: (0, i)),
        ],
        out_shape=[
            jax.ShapeDtypeStruct((K, TC_TOK), jnp.float32),
            jax.ShapeDtypeStruct((K, TC_TOK), jnp.int32),
        ],
    )(wcat, x, eps_t)


@jax.jit
def _run(x, wcat, eps_t):
    s3 = _tc_scores(x, wcat, eps_t)
    pw3, pi3 = _sc_topk(s3)
    pwt, pit = _tc_fused(x, wcat, eps_t)
    pw = jnp.concatenate(
        [pw3.transpose(0, 2, 1).reshape(SC_TOK, K), pwt.T], axis=0
    )
    pi = jnp.concatenate(
        [pi3.transpose(0, 2, 1).reshape(SC_TOK, K), pit.T], axis=0
    )
    return pw, pi


_CONST_CACHE = []


def _consts():
    if not _CONST_CACHE:
        eps_t = jax.jit(
            lambda: jnp.transpose(
                jax.random.normal(jax.random.key(42), (N_TOK, E), dtype=jnp.float32)
            )
        )()
        _CONST_CACHE.append(jax.block_until_ready(eps_t))
    return _CONST_CACHE[0]


def kernel(x, W_gate, W_noise):
    wcat = jnp.concatenate([W_gate, W_noise], axis=0)
    return _run(x, wcat, _consts())


# speedup vs baseline: 1.1696x; 1.0016x over previous
import functools

import jax
import jax.numpy as jnp
from jax import lax
from jax.experimental import pallas as pl
from jax.experimental.pallas import tpu as pltpu
from jax.experimental.pallas import tpu_sc as plsc

N_TOK = 32768
D = 4096
E = 64
K = 8
BLK = 1024

NC = 2
NS = 16
NW = NC * NS
GRP = 16

SC_TOK = 8192
TC_TOK = N_TOK - SC_TOK
CH = SC_TOK // NW
SC_BLOCKS = SC_TOK // BLK
TPB = NW // SC_BLOCKS

NEG_INF = float("-inf")


def _scores_kernel(w_ref, x_ref, eps_ref, s3_ref):
    w = w_ref[...]
    x = x_ref[...]
    s2 = jax.lax.dot_general(
        w, x, (((1,), (1,)), ((), ())), preferred_element_type=jnp.float32
    )
    gate = s2[:E, :]
    noise_std = jax.nn.softplus(s2[E:, :])
    s = gate + noise_std * eps_ref[...]
    for k in range(TPB):
        s3_ref[k] = s[:, k * CH:(k + 1) * CH]


def _tc_scores(x, wcat, eps_t):
    return pl.pallas_call(
        _scores_kernel,
        grid=(SC_BLOCKS,),
        in_specs=[
            pl.BlockSpec((2 * E, D), lambda i: (0, 0)),
            pl.BlockSpec((BLK, D), lambda i: (i, 0)),
            pl.BlockSpec((E, BLK), lambda i: (0, i)),
        ],
        out_specs=pl.BlockSpec((TPB, E, CH), lambda i: (i, 0, 0)),
        out_shape=jax.ShapeDtypeStruct((NW, E, CH), jnp.float32),
    )(wcat, x, eps_t)


@functools.partial(
    pl.kernel,
    out_type=[
        jax.ShapeDtypeStruct((NW, K, CH), jnp.float32),
        jax.ShapeDtypeStruct((NW, K, CH), jnp.int32),
    ],
    mesh=plsc.VectorSubcoreMesh(core_axis_name="c", subcore_axis_name="s"),
    scratch_types=[
        pltpu.VMEM((E, CH), jnp.float32),
        pltpu.VMEM((K, CH), jnp.float32),
        pltpu.VMEM((K, CH), jnp.int32),
    ],
)
def _sc_topk(s3_hbm, pw_hbm, pi_hbm, s_v, w_v, i_v):
    wid = lax.axis_index("s") * NC + lax.axis_index("c")
    pltpu.sync_copy(s3_hbm.at[wid], s_v)
    neg_inf = jnp.full((GRP,), NEG_INF, jnp.float32)
    zero_i = jnp.zeros((GRP,), jnp.int32)
    one_i = jnp.full((GRP,), 1, jnp.int32)

    def group(g, carry):
        sl = pl.ds(g * GRP, GRP)
        sel_lo = zero_i
        sel_hi = zero_i
        vals = []
        idxs = []
        for r in range(K):
            m = neg_inf
            mi = zero_i
            for e in range(E):
                v = s_v[e, sl]
                if r > 0:
                    if e < 32:
                        bit = lax.shift_right_logical(sel_lo, e) & 1
                    else:
                        bit = lax.shift_right_logical(sel_hi, e - 32) & 1
                    v = jnp.where(bit != 0, neg_inf, v)
                gt = v > m
                m = jnp.where(gt, v, m)
                mi = jnp.where(gt, e, mi)
            vals.append(m)
            idxs.append(mi)
            if r < K - 1:
                b = lax.shift_left(one_i, mi & 31)
                sel_lo = sel_lo | jnp.where(mi < 32, b, zero_i)
                sel_hi = sel_hi | jnp.where(mi >= 32, b, zero_i)
        m0 = vals[0]
        es = [jnp.exp(v - m0) for v in vals]
        tot = es[0]
        for t in es[1:]:
            tot = tot + t
        for r in range(K):
            w_v[r, sl] = es[r] / tot
            i_v[r, sl] = idxs[r]
        return carry

    lax.fori_loop(0, CH // GRP, group, 0)
    pltpu.sync_copy(w_v, pw_hbm.at[wid])
    pltpu.sync_copy(i_v, pi_hbm.at[wid])


def _fused_kernel(w_ref, x_ref, eps_ref, pw_ref, pi_ref):
    w = w_ref[...]
    x = x_ref[...]
    s2 = jax.lax.dot_general(
        w, x, (((1,), (1,)), ((), ())), preferred_element_type=jnp.float32
    )
    gate = s2[:E, :]
    noise_std = jax.nn.softplus(s2[E:, :])
    s = gate + noise_std * eps_ref[...]

    iota0 = jax.lax.broadcasted_iota(jnp.int32, (E, BLK), 0)
    vals = []
    idxs = []
    cur = s
    for _ in range(K):
        m = jnp.max(cur, axis=0, keepdims=True)
        idx = jnp.min(jnp.where(cur == m, iota0, E), axis=0, keepdims=True)
        vals.append(m)
        idxs.append(idx)
        cur = jnp.where(iota0 == idx, NEG_INF, cur)
    w8 = jnp.concatenate(vals, axis=0)
    i8 = jnp.concatenate(idxs, axis=0)
    e8 = jnp.exp(w8 - w8[0:1, :])
    p8 = e8 / jnp.sum(e8, axis=0, keepdims=True)
    pw_ref[...] = p8
    pi_ref[...] = i8


def _tc_fused(x, wcat, eps_t):
    off = SC_TOK // BLK
    return pl.pallas_call(
        _fused_kernel,
        grid=(TC_TOK // BLK,),
        in_specs=[
            pl.BlockSpec((2 * E, D), lambda i: (0, 0)),
            pl.BlockSpec((BLK, D), lambda i: (i + off, 0)),
            pl.BlockSpec((E, BLK), lambda i: (0, i + off)),
        ],
        out_specs=[
            pl.BlockSpec((K, BLK), lambda i: (0, i)),
            pl.BlockSpec((K, BLK), lambda i: (0, i)),
        ],
        out_shape=[
            jax.ShapeDtypeStruct((K, TC_TOK), jnp.float32),
            jax.ShapeDtypeStruct((K, TC_TOK), jnp.int32),
        ],
    )(wcat, x, eps_t)


@jax.jit
def _run(x, wcat, eps_t):
    s3 = _tc_scores(x, wcat, eps_t)
    pwt, pit = _tc_fused(x, wcat, eps_t)
    pw3, pi3 = _sc_topk(s3)
    pw = jnp.concatenate(
        [pw3.transpose(0, 2, 1).reshape(SC_TOK, K), pwt.T], axis=0
    )
    pi = jnp.concatenate(
        [pi3.transpose(0, 2, 1).reshape(SC_TOK, K), pit.T], axis=0
    )
    return pw, pi


_CONST_CACHE = []


def _consts():
    if not _CONST_CACHE:
        eps_t = jax.jit(
            lambda: jnp.transpose(
                jax.random.normal(jax.random.key(42), (N_TOK, E), dtype=jnp.float32)
            )
        )()
        _CONST_CACHE.append(jax.block_until_ready(eps_t))
    return _CONST_CACHE[0]


def kernel(x, W_gate, W_noise):
    wcat = jnp.concatenate([W_gate, W_noise], axis=0)
    return _run(x, wcat, _consts())


# speedup vs baseline: 1.1721x; 1.0021x over previous
import functools

import jax
import jax.numpy as jnp
from jax import lax
from jax.experimental import pallas as pl
from jax.experimental.pallas import tpu as pltpu
from jax.experimental.pallas import tpu_sc as plsc

N_TOK = 32768
D = 4096
E = 64
K = 8
BLK = 1024

NC = 2
NS = 16
NW = NC * NS
GRP = 16

SC_TOK = 4096
TC_TOK = N_TOK - SC_TOK
CH = SC_TOK // NW
SC_BLOCKS = SC_TOK // BLK
TPB = NW // SC_BLOCKS

NEG_INF = float("-inf")


def _scores_kernel(w_ref, x_ref, eps_ref, s3_ref):
    w = w_ref[...]
    x = x_ref[...]
    s2 = jax.lax.dot_general(
        w, x, (((1,), (1,)), ((), ())), preferred_element_type=jnp.float32
    )
    gate = s2[:E, :]
    noise_std = jax.nn.softplus(s2[E:, :])
    s = gate + noise_std * eps_ref[...]
    for k in range(TPB):
        s3_ref[k] = s[:, k * CH:(k + 1) * CH]


def _tc_scores(x, wcat, eps_t):
    return pl.pallas_call(
        _scores_kernel,
        grid=(SC_BLOCKS,),
        in_specs=[
            pl.BlockSpec((2 * E, D), lambda i: (0, 0)),
            pl.BlockSpec((BLK, D), lambda i: (i, 0)),
            pl.BlockSpec((E, BLK), lambda i: (0, i)),
        ],
        out_specs=pl.BlockSpec((TPB, E, CH), lambda i: (i, 0, 0)),
        out_shape=jax.ShapeDtypeStruct((NW, E, CH), jnp.float32),
    )(wcat, x, eps_t)


@functools.partial(
    pl.kernel,
    out_type=[
        jax.ShapeDtypeStruct((NW, K, CH), jnp.float32),
        jax.ShapeDtypeStruct((NW, K, CH), jnp.int32),
    ],
    mesh=plsc.VectorSubcoreMesh(core_axis_name="c", subcore_axis_name="s"),
    scratch_types=[
        pltpu.VMEM((E, CH), jnp.float32),
        pltpu.VMEM((K, CH), jnp.float32),
        pltpu.VMEM((K, CH), jnp.int32),
    ],
)
def _sc_topk(s3_hbm, pw_hbm, pi_hbm, s_v, w_v, i_v):
    wid = lax.axis_index("s") * NC + lax.axis_index("c")
    pltpu.sync_copy(s3_hbm.at[wid], s_v)
    neg_inf = jnp.full((GRP,), NEG_INF, jnp.float32)
    zero_i = jnp.zeros((GRP,), jnp.int32)
    one_i = jnp.full((GRP,), 1, jnp.int32)

    def group(g, carry):
        sl = pl.ds(g * GRP, GRP)
        sel_lo = zero_i
        sel_hi = zero_i
        vals = []
        idxs = []
        for r in range(K):
            m = neg_inf
            mi = zero_i
            for e in range(E):
                v = s_v[e, sl]
                if r > 0:
                    if e < 32:
                        bit = lax.shift_right_logical(sel_lo, e) & 1
                    else:
                        bit = lax.shift_right_logical(sel_hi, e - 32) & 1
                    v = jnp.where(bit != 0, neg_inf, v)
                gt = v > m
                m = jnp.where(gt, v, m)
                mi = jnp.where(gt, e, mi)
            vals.append(m)
            idxs.append(mi)
            if r < K - 1:
                b = lax.shift_left(one_i, mi & 31)
                sel_lo = sel_lo | jnp.where(mi < 32, b, zero_i)
                sel_hi = sel_hi | jnp.where(mi >= 32, b, zero_i)
        m0 = vals[0]
        es = [jnp.exp(v - m0) for v in vals]
        tot = es[0]
        for t in es[1:]:
            tot = tot + t
        for r in range(K):
            w_v[r, sl] = es[r] / tot
            i_v[r, sl] = idxs[r]
        return carry

    lax.fori_loop(0, CH // GRP, group, 0)
    pltpu.sync_copy(w_v, pw_hbm.at[wid])
    pltpu.sync_copy(i_v, pi_hbm.at[wid])


def _fused_kernel(w_ref, x_ref, eps_ref, pw_ref, pi_ref):
    w = w_ref[...]
    x = x_ref[...]
    s2 = jax.lax.dot_general(
        w, x, (((1,), (1,)), ((), ())), preferred_element_type=jnp.float32
    )
    gate = s2[:E, :]
    noise_std = jax.nn.softplus(s2[E:, :])
    s = gate + noise_std * eps_ref[...]

    iota0 = jax.lax.broadcasted_iota(jnp.int32, (E, BLK), 0)
    vals = []
    idxs = []
    cur = s
    for _ in range(K):
        m = jnp.max(cur, axis=0, keepdims=True)
        idx = jnp.min(jnp.where(cur == m, iota0, E), axis=0, keepdims=True)
        vals.append(m)
        idxs.append(idx)
        cur = jnp.where(iota0 == idx, NEG_INF, cur)
    w8 = jnp.concatenate(vals, axis=0)
    i8 = jnp.concatenate(idxs, axis=0)
    e8 = jnp.exp(w8 - w8[0:1, :])
    p8 = e8 / jnp.sum(e8, axis=0, keepdims=True)
    pw_ref[...] = p8
    pi_ref[...] = i8


def _tc_fused(x, wcat, eps_t):
    off = SC_TOK // BLK
    return pl.pallas_call(
        _fused_kernel,
        grid=(TC_TOK // BLK,),
        in_specs=[
            pl.BlockSpec((2 * E, D), lambda i: (0, 0)),
            pl.BlockSpec((BLK, D), lambda i: (i + off, 0)),
            pl.BlockSpec((E, BLK), lambda i: (0, i + off)),
        ],
        out_specs=[
            pl.BlockSpec((K, BLK), lambda i: (0, i)),
            pl.BlockSpec((K, BLK), lambda i: (0, i)),
        ],
        out_shape=[
            jax.ShapeDtypeStruct((K, TC_TOK), jnp.float32),
            jax.ShapeDtypeStruct((K, TC_TOK), jnp.int32),
        ],
    )(wcat, x, eps_t)


@jax.jit
def _run(x, wcat, eps_t):
    s3 = _tc_scores(x, wcat, eps_t)
    pwt, pit = _tc_fused(x, wcat, eps_t)
    pw3, pi3 = _sc_topk(s3)
    pw = jnp.concatenate(
        [pw3.transpose(0, 2, 1).reshape(SC_TOK, K), pwt.T], axis=0
    )
    pi = jnp.concatenate(
        [pi3.transpose(0, 2, 1).reshape(SC_TOK, K), pit.T], axis=0
    )
    return pw, pi


_CONST_CACHE = []


def _consts():
    if not _CONST_CACHE:
        eps_t = jax.jit(
            lambda: jnp.transpose(
                jax.random.normal(jax.random.key(42), (N_TOK, E), dtype=jnp.float32)
            )
        )()
        _CONST_CACHE.append(jax.block_until_ready(eps_t))
    return _CONST_CACHE[0]


def kernel(x, W_gate, W_noise):
    wcat = jnp.concatenate([W_gate, W_noise], axis=0)
    return _run(x, wcat, _consts())
